# baseline (speedup 1.0000x reference)
import jax
import jax.numpy as jnp
from jax import lax
from jax.experimental import pallas as pl
from jax.experimental.pallas import tpu as pltpu


def _spike_body(spk_ref, pot_ref, out_ref):
    P = pot_ref[0]
    S = spk_ref[0]
    N, T = P.shape
    f32 = jnp.float32

    mu_col = jnp.max(P, axis=1, keepdims=True)
    sub_i = lax.broadcasted_iota(jnp.int32, (N, N), 0)
    lane_i = lax.broadcasted_iota(jnp.int32, (N, N), 1)
    eye = (sub_i == lane_i).astype(f32)
    ones_row = jnp.ones((1, N), f32)
    mu_row = jnp.dot(ones_row, eye * mu_col, preferred_element_type=f32,
                     precision=lax.Precision.HIGHEST)
    gt = mu_row > mu_col
    tie = (mu_row == mu_col) & (lane_i < sub_i)
    nr_col = 1.0 + jnp.sum((gt | tie).astype(f32), axis=1, keepdims=True)

    RC = 16
    ts = lax.broadcasted_iota(jnp.int32, (RC, T), 1).astype(f32) + 1.0
    mins = []
    for c in range(N // RC):
        p = P[c * RC:(c + 1) * RC]
        s = S[c * RC:(c + 1) * RC]

        def body(d, carry):
            acc, rolled = carry
            acc = acc + (rolled > p).astype(f32)
            rolled = pltpu.roll(rolled, T - 1, 1)
            return acc, rolled

        acc0 = jnp.zeros((RC, T), f32)
        rolled0 = pltpu.roll(p, T - 1, 1)
        acc, _ = lax.fori_loop(0, T - 1, body, (acc0, rolled0))
        x = nr_col[c * RC:(c + 1) * RC] + 0.01 * acc
        fvals = ts * s + (T + x) * (1.0 - s)
        mins.append(jnp.min(fvals, axis=1, keepdims=True))
    mcol = jnp.concatenate(mins, axis=0)
    out_ref[0] = jnp.dot(ones_row, eye * mcol, preferred_element_type=f32,
                         precision=lax.Precision.HIGHEST)


def kernel(output_spikes, output_potentials):
    B, N, T = output_spikes.shape
    out = pl.pallas_call(
        _spike_body,
        grid=(B,),
        in_specs=[
            pl.BlockSpec((1, N, T), lambda b: (b, 0, 0)),
            pl.BlockSpec((1, N, T), lambda b: (b, 0, 0)),
        ],
        out_specs=pl.BlockSpec((1, 1, N), lambda b: (b, 0, 0)),
        out_shape=jax.ShapeDtypeStruct((B, 1, N), jnp.float32),
        compiler_params=pltpu.CompilerParams(
            dimension_semantics=("arbitrary",)),
    )(output_spikes, output_potentials)
    return out.reshape(B, N)


# speedup vs baseline: 21.2947x; 21.2947x over previous
import jax
import jax.numpy as jnp
from jax import lax
from jax.experimental import pallas as pl
from jax.experimental.pallas import tpu as pltpu


def _spike_body(spk_ref, pot_ref, out_ref):
    P = pot_ref[0]
    S = spk_ref[0]
    N, T = P.shape
    f32 = jnp.float32
    hi = lax.Precision.HIGHEST

    sub_i = lax.broadcasted_iota(jnp.int32, (N, N), 0)
    lane_i = lax.broadcasted_iota(jnp.int32, (N, N), 1)
    eyeN = (sub_i == lane_i).astype(f32)
    ones_row = jnp.ones((1, N), f32)

    mu_col = jnp.max(P, axis=1, keepdims=True)
    mu_row = jnp.dot(ones_row, eyeN * mu_col, preferred_element_type=f32,
                     precision=hi)
    gt = mu_row > mu_col
    tie = (mu_row == mu_col) & (lane_i < sub_i)
    nr_col = 1.0 + jnp.sum((gt | tie).astype(f32), axis=1, keepdims=True)

    PT = lax.dot_general(P, eyeN, (((0,), (0,)), ((), ())),
                         preferred_element_type=f32, precision=hi)

    rows = []
    for r in range(N):
        col = PT[:, r:r + 1]
        row = P[r:r + 1, :]
        cnt = jnp.sum((col > row).astype(f32), axis=0, keepdims=True)
        rows.append(cnt)
    TR = jnp.concatenate(rows, axis=0)

    ts = lax.broadcasted_iota(jnp.int32, (N, T), 1).astype(f32) + 1.0
    x = nr_col + 0.01 * TR
    fvals = ts * S + (T + x) * (1.0 - S)
    mcol = jnp.min(fvals, axis=1, keepdims=True)
    out_ref[0] = jnp.dot(ones_row, eyeN * mcol, preferred_element_type=f32,
                         precision=hi)


def kernel(output_spikes, output_potentials):
    B, N, T = output_spikes.shape
    out = pl.pallas_call(
        _spike_body,
        grid=(B,),
        in_specs=[
            pl.BlockSpec((1, N, T), lambda b: (b, 0, 0)),
            pl.BlockSpec((1, N, T), lambda b: (b, 0, 0)),
        ],
        out_specs=pl.BlockSpec((1, 1, N), lambda b: (b, 0, 0)),
        out_shape=jax.ShapeDtypeStruct((B, 1, N), jnp.float32),
        compiler_params=pltpu.CompilerParams(
            dimension_semantics=("arbitrary",)),
    )(output_spikes, output_potentials)
    return out.reshape(B, N)


# speedup vs baseline: 59.9817x; 2.8167x over previous
import functools

import jax
import jax.numpy as jnp
from jax import lax
from jax.experimental import pallas as pl
from jax.experimental.pallas import tpu as pltpu
from jax.experimental.pallas import tpu_sc as plsc

_B, _N, _T = 256, 128, 512
_NTILES = 32
_BPT = _B // _NTILES

_DNUMS = lax.GatherDimensionNumbers(
    offset_dims=(), collapsed_slice_dims=(0,), start_index_map=(0,))


def _shuf(x, idx):
    return lax.gather(x, idx.reshape(16, 1), dimension_numbers=_DNUMS,
                      slice_sizes=(1,),
                      mode=lax.GatherScatterMode.PROMISE_IN_BOUNDS)


def _sc_body(spk_hbm, pot_hbm, out_hbm, pbuf, sbuf, mu_v, nr_v, ab_v,
             cand_v, res_v):
    f32 = jnp.float32
    i32 = jnp.int32
    cid = lax.axis_index("c")
    sid = lax.axis_index("s")
    wid = sid * 2 + cid
    i16 = lax.iota(i32, 16)
    z16 = jnp.zeros((16,), i32)

    def max_splat(x):
        for st in (8, 4, 2, 1):
            x = jnp.maximum(x, _shuf(x, i16 ^ st))
        return x

    def min_splat(x):
        for st in (8, 4, 2, 1):
            x = jnp.minimum(x, _shuf(x, i16 ^ st))
        return x

    def sum_splat(x):
        for st in (8, 4, 2, 1):
            x = x + _shuf(x, i16 ^ st)
        return x

    def prefix_sum(v):
        for st in (1, 2, 4, 8):
            shifted = _shuf(v, jnp.maximum(i16 - st, 0))
            v = v + jnp.where(i16 >= st, shifted, 0)
        return v

    def batch_body(i, _):
        base = (wid * _BPT + i) * _N
        pltpu.sync_copy(pot_hbm.at[pl.ds(base, _N)], pbuf)

        def mu_group(g, _):
            def mu_row(l, accv):
                n = g * 16 + l

                def chunk_max(j, mx):
                    return jnp.maximum(mx, pbuf[n, pl.ds(j * 16, 16)])

                mx = lax.fori_loop(0, _T // 16, chunk_max,
                                   jnp.full((16,), -1e30, f32))
                return jnp.where(i16 == l, max_splat(mx), accv)

            accv = lax.fori_loop(0, 16, mu_row, jnp.zeros((16,), f32))
            mu_v[pl.ds(g * 16, 16)] = accv
            return 0

        lax.fori_loop(0, _N // 16, mu_group, 0)

        def nr_group(g, _):
            u = mu_v[pl.ds(g * 16, 16)]
            gidx = i16 + g * 16

            def nr_j(j, acc):
                w = plsc.load_gather(mu_v, [z16 + j])
                hit = (w > u) | ((w == u) & (j < gidx))
                return acc + jnp.where(hit, 1.0, 0.0)

            acc = lax.fori_loop(0, _N, nr_j, jnp.zeros((16,), f32))
            nr_v[pl.ds(g * 16, 16)] = acc + 1.0
            return 0

        lax.fori_loop(0, _N // 16, nr_group, 0)

        def half_body(h, _):
            pltpu.sync_copy(spk_hbm.at[pl.ds(base + h * 64, 64)], sbuf)

            def row_body(r, _):
                n = h * 64 + r
                nrb = plsc.load_gather(nr_v, [z16 + n])

                def a_chunk(j, mn):
                    sv = sbuf[r, pl.ds(j * 16, 16)]
                    tv = (i16 + (j * 16 + 1)).astype(f32)
                    av = sv * tv + (1.0 - sv) * (512.0 + nrb)
                    ab_v[pl.ds(j * 16, 16)] = av
                    return jnp.minimum(mn, av)

                mnv = lax.fori_loop(0, _T // 16, a_chunk,
                                    jnp.full((16,), 1e30, f32))
                thr = min_splat(mnv) + 5.12

                def cand_chunk(j, off):
                    av = ab_v[pl.ds(j * 16, 16)]
                    msk = av <= thr
                    incl = prefix_sum(jnp.where(msk, 1, 0))
                    pos = off + incl - 1
                    plsc.store_scatter(cand_v, [pos], i16 + j * 16, mask=msk)
                    return off + _shuf(incl, z16 + 15)

                nc = lax.fori_loop(0, _T // 16, cand_chunk, z16)

                def ce_cond(st):
                    return jnp.all(st[0] < nc)

                def ce_body(st):
                    kv, best = st
                    ct = plsc.load_gather(cand_v, [kv])
                    pv = plsc.load_gather(pbuf, [z16 + n, ct])
                    sv = plsc.load_gather(sbuf, [z16 + r, ct])

                    def cnt_chunk(j, acc):
                        pc = pbuf[n, pl.ds(j * 16, 16)]
                        return acc + jnp.where(pc > pv, 1.0, 0.0)

                    accv = lax.fori_loop(0, _T // 16, cnt_chunk,
                                         jnp.zeros((16,), f32))
                    trv = sum_splat(accv)
                    ctf = ct.astype(f32)
                    fv = sv * (ctf + 1.0) + (1.0 - sv) * (
                        512.0 + (nrb + 0.01 * trv))
                    return kv + 1, jnp.minimum(best, fv)

                _, best = lax.while_loop(
                    ce_cond, ce_body, (z16, jnp.full((16,), 1e30, f32)))
                plsc.store_scatter(res_v, [z16 + n], best, mask=i16 == 0)
                return 0

            lax.fori_loop(0, 64, row_body, 0)
            return 0

        lax.fori_loop(0, 2, half_body, 0)
        pltpu.sync_copy(res_v, out_hbm.at[pl.ds(base, _N)])
        return 0

    lax.fori_loop(0, _BPT, batch_body, 0)


def kernel(output_spikes, output_potentials):
    B, N, T = output_spikes.shape
    spk = output_spikes.reshape(B * N, T)
    pot = output_potentials.reshape(B * N, T)
    mesh = plsc.VectorSubcoreMesh(core_axis_name="c", subcore_axis_name="s")
    run = functools.partial(
        pl.kernel,
        out_type=jax.ShapeDtypeStruct((B * N,), jnp.float32),
        mesh=mesh,
        compiler_params=pltpu.CompilerParams(needs_layout_passes=False),
        scratch_types=[
            pltpu.VMEM((N, T), jnp.float32),
            pltpu.VMEM((64, T), jnp.float32),
            pltpu.VMEM((N,), jnp.float32),
            pltpu.VMEM((N,), jnp.float32),
            pltpu.VMEM((T,), jnp.float32),
            pltpu.VMEM((T,), jnp.int32),
            pltpu.VMEM((N,), jnp.float32),
        ],
    )(_sc_body)
    out = run(spk, pot)
    return out.reshape(B, N)
